# baseline (speedup 1.0000x reference)
import functools

import jax
import jax.numpy as jnp
from jax import lax
from jax.experimental import pallas as pl
from jax.experimental.pallas import tpu as pltpu
from jax.experimental.pallas import tpu_sc as plsc

NUM_WORKERS = 32


def _embed(seq_flat, token_table, pe_weight, *, L, E, n_chunks):
    tok_per_w = n_chunks * L
    mesh = plsc.VectorSubcoreMesh(core_axis_name="c", subcore_axis_name="s")

    @functools.partial(
        pl.kernel,
        out_type=jax.ShapeDtypeStruct((seq_flat.shape[0], E), jnp.float32),
        mesh=mesh,
        scratch_types=[
            pltpu.VMEM((L,), jnp.int32),
            pltpu.VMEM((L, E), jnp.float32),
            pltpu.VMEM_SHARED((L, E), jnp.float32),
            pltpu.SemaphoreType.DMA,
        ],
        compiler_params=pltpu.CompilerParams(use_tc_tiling_on_sc=False),
    )
    def k(seq_hbm, table_hbm, pe_hbm, out_hbm, idx_v, buf, pe_v, sem):
        c = lax.axis_index("c")
        s = lax.axis_index("s")
        wid = s * 2 + c
        base = wid * tok_per_w

        @pl.when(s == 0)
        def _():
            pltpu.sync_copy(pe_hbm, pe_v)

        plsc.subcore_barrier()

        def body(i, carry):
            off = base + i * L
            pltpu.sync_copy(seq_hbm.at[pl.ds(off, L)], idx_v)
            pltpu.sync_copy(pe_v, buf)
            pltpu.async_copy(table_hbm.at[idx_v], buf, sem, add=True).wait()
            pltpu.sync_copy(buf, out_hbm.at[pl.ds(off, L)])
            return carry

        lax.fori_loop(0, n_chunks, body, 0)

    return k(seq_flat, token_table, pe_weight)


def kernel(sequence, token_table, pe_weight):
    B, L = sequence.shape
    V, E = token_table.shape
    seq_flat = sequence.reshape(B * L).astype(jnp.int32)
    n_chunks = (B * L) // (NUM_WORKERS * L)
    out = _embed(seq_flat, token_table, pe_weight, L=L, E=E, n_chunks=n_chunks)
    return out.reshape(B, L, E)


# speedup vs baseline: 1.1688x; 1.1688x over previous
import functools

import jax
import jax.numpy as jnp
from jax import lax
from jax.experimental import pallas as pl
from jax.experimental.pallas import tpu as pltpu
from jax.experimental.pallas import tpu_sc as plsc

NUM_WORKERS = 32
SEQ_PER_CHUNK = 2


def _embed(seq_flat, token_table, pe_weight, *, L, E):
    N = seq_flat.shape[0]
    tok_per_w = N // NUM_WORKERS
    C = SEQ_PER_CHUNK * L
    n_pairs = tok_per_w // (2 * C)
    mesh = plsc.VectorSubcoreMesh(core_axis_name="c", subcore_axis_name="s")

    @functools.partial(
        pl.kernel,
        out_type=jax.ShapeDtypeStruct((N, E), jnp.float32),
        mesh=mesh,
        scratch_types=[
            pltpu.VMEM((tok_per_w,), jnp.int32),
            pltpu.VMEM((C, E), jnp.float32),
            pltpu.VMEM((C, E), jnp.float32),
            pltpu.VMEM_SHARED((C, E), jnp.float32),
            pltpu.SemaphoreType.DMA,
            pltpu.SemaphoreType.DMA,
            pltpu.SemaphoreType.DMA,
            pltpu.SemaphoreType.DMA,
            pltpu.SemaphoreType.DMA,
            pltpu.SemaphoreType.DMA,
            pltpu.SemaphoreType.DMA,
        ],
        compiler_params=pltpu.CompilerParams(use_tc_tiling_on_sc=False),
    )
    def k(seq_hbm, table_hbm, pe_hbm, out_hbm,
          idx_v, buf0, buf1, pe_sh,
          sem_idx, pre0, pre1, g0, g1, wb0, wb1):
        c = lax.axis_index("c")
        s = lax.axis_index("s")
        wid = s * 2 + c
        base = wid * tok_per_w

        idx_cp = pltpu.async_copy(seq_hbm.at[pl.ds(base, tok_per_w)],
                                  idx_v, sem_idx)

        @pl.when(s == 0)
        def _():
            for r in range(SEQ_PER_CHUNK):
                pltpu.sync_copy(pe_hbm, pe_sh.at[pl.ds(r * L, L)])

        plsc.subcore_barrier()

        pltpu.async_copy(pe_sh, buf0, pre0)
        pltpu.async_copy(pe_sh, buf1, pre1)
        idx_cp.wait()

        def gather(buf, sem, off):
            return pltpu.async_copy(
                table_hbm.at[idx_v.at[pl.ds(off, C)]], buf, sem, add=True)

        def body(j, carry):
            off0 = 2 * j * C
            off1 = off0 + C
            pltpu.make_async_copy(pe_sh, buf0, pre0).wait()
            gather(buf0, g0, off0)
            @pl.when(j > 0)
            def _():
                pltpu.make_async_copy(buf1, out_hbm.at[pl.ds(base, C)],
                                      wb1).wait()
                pltpu.async_copy(pe_sh, buf1, pre1)
            pltpu.make_async_copy(
                table_hbm.at[idx_v.at[pl.ds(off0, C)]], buf0, g0).wait()
            pltpu.async_copy(buf0, out_hbm.at[pl.ds(base + off0, C)], wb0)
            pltpu.make_async_copy(pe_sh, buf1, pre1).wait()
            gather(buf1, g1, off1)
            pltpu.make_async_copy(buf0, out_hbm.at[pl.ds(base, C)],
                                  wb0).wait()
            pltpu.async_copy(pe_sh, buf0, pre0)
            pltpu.make_async_copy(
                table_hbm.at[idx_v.at[pl.ds(off1, C)]], buf1, g1).wait()
            pltpu.async_copy(buf1, out_hbm.at[pl.ds(base + off1, C)], wb1)
            return carry

        lax.fori_loop(0, n_pairs, body, 0, unroll=False)

        pltpu.make_async_copy(pe_sh, buf0, pre0).wait()
        pltpu.make_async_copy(buf1, out_hbm.at[pl.ds(base, C)], wb1).wait()

    return k(seq_flat, token_table, pe_weight)


def kernel(sequence, token_table, pe_weight):
    B, L = sequence.shape
    V, E = token_table.shape
    seq_flat = sequence.reshape(B * L).astype(jnp.int32)
    out = _embed(seq_flat, token_table, pe_weight, L=L, E=E)
    return out.reshape(B, L, E)


# speedup vs baseline: 1.5084x; 1.2905x over previous
import functools

import jax
import jax.numpy as jnp
from jax import lax
from jax.experimental import pallas as pl
from jax.experimental.pallas import tpu as pltpu
from jax.experimental.pallas import tpu_sc as plsc

NUM_WORKERS = 32


def _embed(seq_flat, token_table, pe_weight, *, B, L, E):
    N = seq_flat.shape[0]
    tok_per_w = N // NUM_WORKERS
    C = L
    seq_per_w = B // NUM_WORKERS
    n_pairs = seq_per_w // 2
    mesh = plsc.VectorSubcoreMesh(core_axis_name="c", subcore_axis_name="s")

    @functools.partial(
        pl.kernel,
        out_type=jax.ShapeDtypeStruct((B, L, 2 * E), jnp.float32),
        mesh=mesh,
        scratch_types=[
            pltpu.VMEM((tok_per_w,), jnp.int32),
            pltpu.VMEM((C, E), jnp.float32),
            pltpu.VMEM((C, E), jnp.float32),
            pltpu.VMEM_SHARED((C, E), jnp.float32),
            pltpu.SemaphoreType.DMA,
            pltpu.SemaphoreType.DMA,
            pltpu.SemaphoreType.DMA,
            pltpu.SemaphoreType.DMA,
            pltpu.SemaphoreType.DMA,
            pltpu.SemaphoreType.DMA,
            pltpu.SemaphoreType.DMA,
        ],
        compiler_params=pltpu.CompilerParams(use_tc_tiling_on_sc=False),
    )
    def k(seq_hbm, table_hbm, pe_hbm, out_hbm,
          idx_v, buf0, buf1, pe_sh,
          sem_idx, pre0, pre1, g0, g1, wb0, wb1):
        c = lax.axis_index("c")
        s = lax.axis_index("s")
        wid = s * 2 + c
        base = wid * tok_per_w
        seq_base = wid * seq_per_w

        idx_cp = pltpu.async_copy(seq_hbm.at[pl.ds(base, tok_per_w)],
                                  idx_v, sem_idx)

        @pl.when(s == 0)
        def _():
            pltpu.sync_copy(pe_hbm, pe_sh)

        plsc.subcore_barrier()

        pltpu.async_copy(pe_sh, buf0, pre0)
        pltpu.async_copy(pe_sh, buf1, pre1)
        idx_cp.wait()

        def gather(buf, sem, off):
            return pltpu.async_copy(
                table_hbm.at[idx_v.at[pl.ds(off, C)]], buf, sem, add=True)

        def wb_copy(buf, sem, seq_i):
            return pltpu.make_async_copy(
                buf, out_hbm.at[seq_base + seq_i].at[:, pl.ds(0, E)], sem)

        def body(j, carry):
            off0 = 2 * j * C
            off1 = off0 + C
            pltpu.make_async_copy(pe_sh, buf0, pre0).wait()
            gather(buf0, g0, off0)
            @pl.when(j > 0)
            def _():
                wb_copy(buf1, wb1, 0).wait()
                pltpu.async_copy(pe_sh, buf1, pre1)
            pltpu.make_async_copy(
                table_hbm.at[idx_v.at[pl.ds(off0, C)]], buf0, g0).wait()
            wb_copy(buf0, wb0, 2 * j).start()
            pltpu.make_async_copy(pe_sh, buf1, pre1).wait()
            gather(buf1, g1, off1)
            wb_copy(buf0, wb0, 0).wait()
            pltpu.async_copy(pe_sh, buf0, pre0)
            pltpu.make_async_copy(
                table_hbm.at[idx_v.at[pl.ds(off1, C)]], buf1, g1).wait()
            wb_copy(buf1, wb1, 2 * j + 1).start()
            return carry

        lax.fori_loop(0, n_pairs, body, 0, unroll=False)

        pltpu.make_async_copy(pe_sh, buf0, pre0).wait()
        wb_copy(buf1, wb1, 0).wait()

    return k(seq_flat, token_table, pe_weight)


def kernel(sequence, token_table, pe_weight):
    B, L = sequence.shape
    V, E = token_table.shape
    seq_flat = sequence.reshape(B * L).astype(jnp.int32)
    out = _embed(seq_flat, token_table, pe_weight, B=B, L=L, E=E)
    return out[:, :, :E]
